# baseline (speedup 1.0000x reference)
import functools

import jax
import jax.numpy as jnp
from jax import lax
from jax.experimental import pallas as pl
from jax.experimental.pallas import tpu as pltpu
from jax.experimental.pallas import tpu_sc as plsc

VOCAB = 50257
MAX_POS = 2048
D = 1024
B = 4
S = 2048

NC = 2
NS = 16
NW = NC * NS
PPW = S // NW
K = 16
NH = PPW // K
NCH = NH * B
NBUF = 4
LANES = 16
CPR = D // LANES
UNROLL = 1

_mesh = plsc.VectorSubcoreMesh(core_axis_name="c", subcore_axis_name="s")


@functools.partial(
    pl.kernel,
    mesh=_mesh,
    out_type=jax.ShapeDtypeStruct((B * S, D), jnp.float32),
    scratch_types=[
        pltpu.VMEM((NCH, K), jnp.int32),
        pltpu.VMEM((K, D), jnp.float32),
        pltpu.VMEM((K, D), jnp.float32),
        pltpu.VMEM((K, D), jnp.float32),
        pltpu.VMEM((K, D), jnp.float32),
        pltpu.VMEM((K, D), jnp.float32),
        pltpu.SemaphoreType.DMA,
        pltpu.SemaphoreType.DMA,
        pltpu.SemaphoreType.DMA,
        pltpu.SemaphoreType.DMA,
        pltpu.SemaphoreType.DMA,
        pltpu.SemaphoreType.DMA,
        pltpu.SemaphoreType.DMA,
        pltpu.SemaphoreType.DMA,
    ],
)
def _emb_kernel(ids_hbm, wte_hbm, wpe_hbm, out_hbm,
                idx_v, pbuf, gbuf0, gbuf1, gbuf2, gbuf3,
                gs0, gs1, gs2, gs3, ws0, ws1, ws2, ws3):
    cid = lax.axis_index("c")
    sid = lax.axis_index("s")
    wid = sid * NC + cid
    pos_base = pl.multiple_of(wid * PPW, PPW)

    gb = (gbuf0, gbuf1, gbuf2, gbuf3)
    gs = (gs0, gs1, gs2, gs3)
    ws = (ws0, ws1, ws2, ws3)

    def gather_to(j, buf):
        return pltpu.async_copy(wte_hbm.at[idx_v.at[j]], gb[buf], gs[buf])

    def drain_gather(buf):
        pltpu.make_async_copy(
            wte_hbm.at[idx_v.at[0]], gb[buf], gs[buf]).wait()

    def start_write(j, buf):
        h = j // B
        b = j - h * B
        row0 = pl.multiple_of(b * S + pos_base + h * K, K)
        return pltpu.async_copy(gb[buf], out_hbm.at[pl.ds(row0, K)], ws[buf])

    def drain_write(buf):
        pltpu.make_async_copy(
            gb[buf], out_hbm.at[pl.ds(0, K)], ws[buf]).wait()

    def stage_quarter(j):
        off = pl.multiple_of(pos_base + (j // B) * K, K)
        pltpu.sync_copy(wpe_hbm.at[pl.ds(off, K)], pbuf)

    def add_chunk(buf):
        g = gb[buf]

        @plsc.parallel_loop(0, K, unroll=UNROLL)
        def add_row(r):
            for c in range(CPR):
                v = pbuf[r, pl.ds(c * LANES, LANES)]
                g[r, pl.ds(c * LANES, LANES)] = g[r, pl.ds(c * LANES, LANES)] + v

    id_copies = []
    for j in range(NCH):
        h, b = divmod(j, B)
        src = ids_hbm.at[pl.ds(pl.multiple_of(b * S + pos_base + h * K, K), K)]
        id_copies.append(pltpu.async_copy(src, idx_v.at[j], gs0))
    for c in id_copies:
        c.wait()
    gather_to(0, 0)

    def body(j, carry):
        for buf in range(NBUF):

            @pl.when(j % NBUF == buf)
            def _():
                nxt = (buf + 1) % NBUF

                @pl.when(j >= NBUF - 1)
                def _():
                    drain_write(nxt)

                @pl.when(j < NCH - 1)
                def _():
                    gather_to(j + 1, nxt)

                drain_gather(buf)

                @pl.when(j % B == 0)
                def _():
                    stage_quarter(j)

                add_chunk(buf)
                start_write(j, buf)

        return carry

    lax.fori_loop(0, NCH, body, 0)

    for j in range(NCH - (NBUF - 1), NCH):
        drain_write(j % NBUF)


def kernel(input_ids, wte, wpe):
    out = _emb_kernel(input_ids.astype(jnp.int32).reshape(B * S), wte, wpe)
    return out.reshape(B, S, D)


# speedup vs baseline: 1.0170x; 1.0170x over previous
import functools

import jax
import jax.numpy as jnp
from jax import lax
from jax.experimental import pallas as pl
from jax.experimental.pallas import tpu as pltpu
from jax.experimental.pallas import tpu_sc as plsc

VOCAB = 50257
MAX_POS = 2048
D = 1024
B = 4
S = 2048

NC = 2
NS = 16
NW = NC * NS
PPW = S // NW
K = 16
NH = PPW // K
NCH = NH * B
NBUF = 3
LANES = 16
CPR = D // LANES
UNROLL = 1

_mesh = plsc.VectorSubcoreMesh(core_axis_name="c", subcore_axis_name="s")


@functools.partial(
    pl.kernel,
    mesh=_mesh,
    out_type=jax.ShapeDtypeStruct((B * S, D), jnp.float32),
    scratch_types=[
        pltpu.VMEM((NCH, K), jnp.int32),
        pltpu.VMEM((K, D), jnp.float32),
        pltpu.VMEM((K, D), jnp.float32),
        pltpu.VMEM((K, D), jnp.float32),
        pltpu.VMEM((K, D), jnp.float32),
        pltpu.SemaphoreType.DMA,
        pltpu.SemaphoreType.DMA,
        pltpu.SemaphoreType.DMA,
        pltpu.SemaphoreType.DMA,
        pltpu.SemaphoreType.DMA,
        pltpu.SemaphoreType.DMA,
    ],
)
def _emb_kernel(ids_hbm, wte_hbm, wpe_hbm, out_hbm,
                idx_v, pbuf, gbuf0, gbuf1, gbuf2,
                gs0, gs1, gs2, ws0, ws1, ws2):
    cid = lax.axis_index("c")
    sid = lax.axis_index("s")
    wid = sid * NC + cid
    pos_base = pl.multiple_of(wid * PPW, PPW)

    gb = (gbuf0, gbuf1, gbuf2)
    gs = (gs0, gs1, gs2)
    ws = (ws0, ws1, ws2)

    def gather_to(j, buf):
        return pltpu.async_copy(wte_hbm.at[idx_v.at[j]], gb[buf], gs[buf])

    def drain_gather(buf):
        pltpu.make_async_copy(
            wte_hbm.at[idx_v.at[0]], gb[buf], gs[buf]).wait()

    def start_write(j, buf):
        h = j // B
        b = j - h * B
        row0 = pl.multiple_of(b * S + pos_base + h * K, K)
        return pltpu.async_copy(gb[buf], out_hbm.at[pl.ds(row0, K)], ws[buf])

    def drain_write(buf):
        pltpu.make_async_copy(
            gb[buf], out_hbm.at[pl.ds(0, K)], ws[buf]).wait()

    def stage_quarter(j):
        off = pl.multiple_of(pos_base + (j // B) * K, K)
        pltpu.sync_copy(wpe_hbm.at[pl.ds(off, K)], pbuf)

    def add_chunk(buf):
        g = gb[buf]

        @plsc.parallel_loop(0, K, unroll=UNROLL)
        def add_row(r):
            for c in range(CPR):
                v = pbuf[r, pl.ds(c * LANES, LANES)]
                g[r, pl.ds(c * LANES, LANES)] = g[r, pl.ds(c * LANES, LANES)] + v

    id_copies = []
    for j in range(NCH):
        h, b = divmod(j, B)
        src = ids_hbm.at[pl.ds(pl.multiple_of(b * S + pos_base + h * K, K), K)]
        id_copies.append(pltpu.async_copy(src, idx_v.at[j], gs0))
    for c in id_copies:
        c.wait()
    gather_to(0, 0)

    def body(j, carry):
        for buf in range(NBUF):

            @pl.when(j % NBUF == buf)
            def _():
                nxt = (buf + 1) % NBUF

                @pl.when(j >= NBUF - 1)
                def _():
                    drain_write(nxt)

                @pl.when(j < NCH - 1)
                def _():
                    gather_to(j + 1, nxt)

                drain_gather(buf)

                @pl.when(j % B == 0)
                def _():
                    stage_quarter(j)

                add_chunk(buf)
                start_write(j, buf)

        return carry

    lax.fori_loop(0, NCH, body, 0)

    for j in range(NCH - (NBUF - 1), NCH):
        drain_write(j % NBUF)


def kernel(input_ids, wte, wpe):
    out = _emb_kernel(input_ids.astype(jnp.int32).reshape(B * S), wte, wpe)
    return out.reshape(B, S, D)


# speedup vs baseline: 1.0686x; 1.0508x over previous
import functools

import jax
import jax.numpy as jnp
from jax import lax
from jax.experimental import pallas as pl
from jax.experimental.pallas import tpu as pltpu
from jax.experimental.pallas import tpu_sc as plsc

VOCAB = 50257
MAX_POS = 2048
D = 1024
B = 4
S = 2048

NC = 2
NS = 16
NW = NC * NS
PPW = S // NW
K = 16
NH = PPW // K
NCH = NH * B
NBUF = 3
LANES = 16
CPR = D // LANES
UNROLL = 1

_mesh = plsc.VectorSubcoreMesh(core_axis_name="c", subcore_axis_name="s")


@functools.partial(
    pl.kernel,
    mesh=_mesh,
    out_type=jax.ShapeDtypeStruct((B * S, D), jnp.float32),
    scratch_types=[
        pltpu.VMEM((NCH, K), jnp.int32),
        pltpu.VMEM((K, D), jnp.float32),
        pltpu.VMEM((K, D), jnp.float32),
        pltpu.SemaphoreType.DMA,
        pltpu.VMEM((K, D), jnp.float32),
        pltpu.VMEM((K, D), jnp.float32),
        pltpu.VMEM((K, D), jnp.float32),
        pltpu.SemaphoreType.DMA,
        pltpu.SemaphoreType.DMA,
        pltpu.SemaphoreType.DMA,
        pltpu.SemaphoreType.DMA,
        pltpu.SemaphoreType.DMA,
        pltpu.SemaphoreType.DMA,
        pltpu.SemaphoreType.DMA,
    ],
)
def _emb_kernel(ids_hbm, wte_hbm, wpe_hbm, out_hbm,
                idx_v, pbufa, pbufb, ps0, gbuf0, gbuf1, gbuf2,
                gs0, gs1, gs2, ws0, ws1, ws2, ps1):
    cid = lax.axis_index("c")
    sid = lax.axis_index("s")
    wid = sid * NC + cid
    pos_base = pl.multiple_of(wid * PPW, PPW)

    pb = (pbufa, pbufb)
    psem = (ps0, ps1)
    gb = (gbuf0, gbuf1, gbuf2)
    gs = (gs0, gs1, gs2)
    ws = (ws0, ws1, ws2)

    def gather_to(j, buf):
        return pltpu.async_copy(wte_hbm.at[idx_v.at[j]], gb[buf], gs[buf])

    def drain_gather(buf):
        pltpu.make_async_copy(
            wte_hbm.at[idx_v.at[0]], gb[buf], gs[buf]).wait()

    def start_write(j, buf):
        h = j // B
        b = j - h * B
        row0 = pl.multiple_of(b * S + pos_base + h * K, K)
        return pltpu.async_copy(gb[buf], out_hbm.at[pl.ds(row0, K)], ws[buf])

    def drain_write(buf):
        pltpu.make_async_copy(
            gb[buf], out_hbm.at[pl.ds(0, K)], ws[buf]).wait()

    def stage_quarter(h, p):
        off = pl.multiple_of(pos_base + h * K, K)
        return pltpu.async_copy(wpe_hbm.at[pl.ds(off, K)], pb[p], psem[p])

    def drain_stage(p):
        pltpu.make_async_copy(
            wpe_hbm.at[pl.ds(0, K)], pb[p], psem[p]).wait()

    def add_chunk(buf, p):
        g = gb[buf]
        pbuf = pb[p]

        @plsc.parallel_loop(0, K, unroll=UNROLL)
        def add_row(r):
            for c in range(CPR):
                v = pbuf[r, pl.ds(c * LANES, LANES)]
                g[r, pl.ds(c * LANES, LANES)] = g[r, pl.ds(c * LANES, LANES)] + v

    id_copies = []
    for j in range(NCH):
        h, b = divmod(j, B)
        src = ids_hbm.at[pl.ds(pl.multiple_of(b * S + pos_base + h * K, K), K)]
        id_copies.append(pltpu.async_copy(src, idx_v.at[j], gs0))
    for c in id_copies:
        c.wait()
    gather_to(0, 0)
    stage_quarter(0, 0)

    def body(j, carry):
        for buf in range(NBUF):

            @pl.when(j % NBUF == buf)
            def _():
                nxt = (buf + 1) % NBUF

                @pl.when(j >= NBUF - 1)
                def _():
                    drain_write(nxt)

                @pl.when(j < NCH - 1)
                def _():
                    gather_to(j + 1, nxt)

                drain_gather(buf)

                for p in range(2):

                    @pl.when(((j // B) & 1) == p)
                    def _():

                        @pl.when(j % B == 0)
                        def _():
                            drain_stage(p)

                            @pl.when(j < (NH - 1) * B)
                            def _():
                                stage_quarter(j // B + 1, 1 - p)

                        add_chunk(buf, p)

                start_write(j, buf)

        return carry

    lax.fori_loop(0, NCH, body, 0)

    for j in range(NCH - (NBUF - 1), NCH):
        drain_write(j % NBUF)


def kernel(input_ids, wte, wpe):
    out = _emb_kernel(input_ids.astype(jnp.int32).reshape(B * S), wte, wpe)
    return out.reshape(B, S, D)
